# baseline (speedup 1.0000x reference)
import functools

import jax
import jax.numpy as jnp
from jax import lax
from jax.experimental import pallas as pl
from jax.experimental.pallas import tpu as pltpu
from jax.experimental.pallas import tpu_sc as plsc

_N_FIELDS = 26
_EMB = 64
_DM = 512
_LANES = 128
_STEPS = _N_FIELDS + 1


def _sc_gather(idxs32, uid_t, cat_t):
    mesh = plsc.VectorSubcoreMesh(core_axis_name="c", subcore_axis_name="s")
    info = plsc.get_sparse_core_info()
    nc = info.num_cores

    @functools.partial(
        pl.kernel,
        mesh=mesh,
        out_type=jax.ShapeDtypeStruct((_STEPS * _EMB,), jnp.float32),
        scratch_types=[
            pltpu.VMEM((32,), jnp.int32),
            pltpu.VMEM((_EMB, _LANES), jnp.float32),
            pltpu.VMEM((_EMB,), jnp.float32),
        ],
        compiler_params=pltpu.CompilerParams(use_tc_tiling_on_sc=True,
                                             needs_layout_passes=False),
    )
    def k(idxs_hbm, uid_hbm, cat_hbm, out_hbm, idx_v, tile_v, col_v):
        w = lax.axis_index("s") * nc + lax.axis_index("c")

        @pl.when(w < _STEPS)
        def _work():
            pltpu.sync_copy(idxs_hbm, idx_v)
            lanes16 = lax.broadcasted_iota(jnp.int32, (16,), 0)
            lo = jnp.where(lanes16 == w, idx_v[pl.ds(0, 16)], 0)
            hi = jnp.where(lanes16 == w - 16, idx_v[pl.ds(16, 16)], 0)
            idx = jnp.sum(lo, axis=0) + jnp.sum(hi, axis=0)
            base = (idx // _LANES) * _LANES
            lane = idx % _LANES

            @pl.when(w == 0)
            def _uid():
                pltpu.sync_copy(uid_hbm.at[:, pl.ds(base, _LANES)], tile_v)

            @pl.when(w > 0)
            def _cat():
                pltpu.sync_copy(cat_hbm.at[w - 1, :, pl.ds(base, _LANES)],
                                tile_v)

            lane_vec = jnp.full((16,), lane, jnp.int32)
            for g in range(_EMB // 16):
                rows = lax.broadcasted_iota(jnp.int32, (16,), 0) + g * 16
                col_v[pl.ds(g * 16, 16)] = plsc.load_gather(
                    tile_v, [rows, lane_vec])
            pltpu.sync_copy(col_v, out_hbm.at[pl.ds(w * _EMB, _EMB)])

    return k(idxs32, uid_t, cat_t)


def _tc_mlp_body(fu_hbm, w1_hbm, b1_hbm, w2_hbm, b2_hbm, out_ref, fu_ref,
                 w1_ref, b1_ref, w2_ref, b2_ref, sems):
    w1_dma = pltpu.make_async_copy(w1_hbm, w1_ref, sems.at[0])
    w2_dma = pltpu.make_async_copy(w2_hbm, w2_ref, sems.at[1])
    b1_dma = pltpu.make_async_copy(b1_hbm, b1_ref, sems.at[2])
    b2_dma = pltpu.make_async_copy(b2_hbm, b2_ref, sems.at[3])
    fu_dma = pltpu.make_async_copy(fu_hbm, fu_ref, sems.at[4])
    w1_dma.start()
    fu_dma.start()
    w2_dma.start()
    b1_dma.start()
    b2_dma.start()

    fu_dma.wait()
    w1_dma.wait()
    b1_dma.wait()
    acc = None
    for s in range(_STEPS):
        row = fu_ref[pl.ds(s * _EMB, _EMB)].reshape(1, _EMB)
        partial = jnp.dot(row, w1_ref[pl.ds(s * _EMB, _EMB), :],
                          preferred_element_type=jnp.float32)
        acc = partial if acc is None else acc + partial
    x = acc + b1_ref[...]
    x = jnp.where(x >= 0, x, 0.01 * x)
    w2_dma.wait()
    b2_dma.wait()
    out_ref[...] = (jnp.dot(x, w2_ref[...], preferred_element_type=jnp.float32)
                    + b2_ref[...])


def kernel(uid, onehot_feats, uid_table, cat_tables, W1, b1, W2, b2):
    uid_t = uid_table.T
    cat_t = jnp.transpose(cat_tables, (0, 2, 1))
    idxs32 = jnp.concatenate(
        [uid.astype(jnp.int32), onehot_feats.reshape(-1).astype(jnp.int32),
         jnp.zeros((32 - _STEPS,), jnp.int32)])

    fu = _sc_gather(idxs32, uid_t, cat_t)

    out = pl.pallas_call(
        _tc_mlp_body,
        in_specs=[
            pl.BlockSpec(memory_space=pl.ANY),
            pl.BlockSpec(memory_space=pl.ANY),
            pl.BlockSpec(memory_space=pl.ANY),
            pl.BlockSpec(memory_space=pl.ANY),
            pl.BlockSpec(memory_space=pl.ANY),
        ],
        out_specs=pl.BlockSpec(memory_space=pltpu.VMEM),
        out_shape=jax.ShapeDtypeStruct((1, _DM), jnp.float32),
        scratch_shapes=[
            pltpu.VMEM((_STEPS * _EMB,), jnp.float32),
            pltpu.VMEM((_STEPS * _EMB, _DM), jnp.float32),
            pltpu.VMEM((1, _DM), jnp.float32),
            pltpu.VMEM((_DM, _DM), jnp.float32),
            pltpu.VMEM((1, _DM), jnp.float32),
            pltpu.SemaphoreType.DMA((5,)),
        ],
    )(fu, W1, b1.reshape(1, -1), W2, b2.reshape(1, -1))
    return out[None]


# speedup vs baseline: 3.2207x; 3.2207x over previous
import jax
import jax.numpy as jnp
from jax.experimental import pallas as pl
from jax.experimental.pallas import tpu as pltpu

_N_FIELDS = 26
_EMB = 64
_DM = 512
_LANES = 128
_STEPS = _N_FIELDS + 1


def _mlp_body(uid_ref, feats_ref, uid_hbm, cat_hbm, w1_hbm, b1_hbm, w2_hbm,
              b2_hbm, out_ref, emb_ref, fu_ref, w1_ref, b1_ref, w2_ref,
              b2_ref, sems, wsems):
    def _idx(s):
        return uid_ref[0] if s == 0 else feats_ref[s - 1, 0]

    w1_dma = pltpu.make_async_copy(w1_hbm, w1_ref, wsems.at[0])
    w2_dma = pltpu.make_async_copy(w2_hbm, w2_ref, wsems.at[1])
    b1_dma = pltpu.make_async_copy(b1_hbm, b1_ref, wsems.at[2])
    b2_dma = pltpu.make_async_copy(b2_hbm, b2_ref, wsems.at[3])
    base0 = (_idx(0) // _LANES) * _LANES
    pltpu.make_async_copy(
        uid_hbm.at[:, pl.ds(base0, _LANES)], emb_ref.at[0], sems.at[0]).start()
    for s in range(1, _STEPS):
        base = (_idx(s) // _LANES) * _LANES
        pltpu.make_async_copy(
            cat_hbm.at[s - 1, :, pl.ds(base, _LANES)], emb_ref.at[s],
            sems.at[s]).start()

    w1_dma.start()
    b1_dma.start()
    w2_dma.start()
    b2_dma.start()

    lane_iota = jax.lax.broadcasted_iota(jnp.int32, (_EMB, _LANES), 1)
    for s in range(_STEPS):
        pltpu.make_async_copy(
            uid_hbm.at[:, pl.ds(0, _LANES)], emb_ref.at[s], sems.at[s]).wait()
        lane = _idx(s) % _LANES
        tile = emb_ref[s]
        col = jnp.sum(jnp.where(lane_iota == lane, tile, 0.0), axis=1,
                      keepdims=True)
        fu_ref[pl.ds(s * _EMB, _EMB), :] = col

    w1_dma.wait()
    b1_dma.wait()
    x = jax.lax.dot_general(
        fu_ref[...], w1_ref[...], (((0,), (0,)), ((), ())),
        preferred_element_type=jnp.float32) + b1_ref[...]
    x = jnp.where(x >= 0, x, 0.01 * x)
    w2_dma.wait()
    b2_dma.wait()
    out_ref[...] = (jnp.dot(x, w2_ref[...], preferred_element_type=jnp.float32)
                    + b2_ref[...])


def kernel(uid, onehot_feats, uid_table, cat_tables, W1, b1, W2, b2):
    uid_t = uid_table.T
    cat_t = jnp.transpose(cat_tables, (0, 2, 1))

    out = pl.pallas_call(
        _mlp_body,
        in_specs=[
            pl.BlockSpec(memory_space=pltpu.SMEM),
            pl.BlockSpec(memory_space=pltpu.SMEM),
            pl.BlockSpec(memory_space=pl.ANY),
            pl.BlockSpec(memory_space=pl.ANY),
            pl.BlockSpec(memory_space=pl.ANY),
            pl.BlockSpec(memory_space=pl.ANY),
            pl.BlockSpec(memory_space=pl.ANY),
            pl.BlockSpec(memory_space=pl.ANY),
        ],
        out_specs=pl.BlockSpec(memory_space=pltpu.VMEM),
        out_shape=jax.ShapeDtypeStruct((1, _DM), jnp.float32),
        scratch_shapes=[
            pltpu.VMEM((_STEPS, _EMB, _LANES), jnp.float32),
            pltpu.VMEM((_STEPS * _EMB, 1), jnp.float32),
            pltpu.VMEM((_STEPS * _EMB, _DM), jnp.float32),
            pltpu.VMEM((1, _DM), jnp.float32),
            pltpu.VMEM((_DM, _DM), jnp.float32),
            pltpu.VMEM((1, _DM), jnp.float32),
            pltpu.SemaphoreType.DMA((_STEPS,)),
            pltpu.SemaphoreType.DMA((4,)),
        ],
    )(uid.astype(jnp.int32), onehot_feats.astype(jnp.int32), uid_t, cat_t,
      W1, b1.reshape(1, -1), W2, b2.reshape(1, -1))
    return out[None]
